# baseline (speedup 1.0000x reference)
import functools

import jax
import jax.numpy as jnp
from jax import lax
from jax.experimental import pallas as pl
from jax.experimental.pallas import tpu as pltpu
from jax.experimental.pallas import tpu_sc as plsc

_B, _T, _D, _CD, _K = 16, 1024, 768, 256, 8192
_N = _B * _T
_S = 4
_H = _N // _S
_R1 = 512
_GH = _H // _R1


def _front_body(feat, w1, b1, w2, b2, g, beta, cbt, idx_out, enc_out, e2_scr):
    @pl.when(pl.program_id(0) == 0)
    def _():
        c = cbt[...]
        e2_scr[...] = jnp.sum(c * c, axis=0, keepdims=True)

    x = feat[...]
    h = jnp.maximum(jnp.dot(x, w1[...], preferred_element_type=jnp.float32)
                    + b1[...], 0.0)
    h = jnp.dot(h, w2[...], preferred_element_type=jnp.float32) + b2[...]
    mu = jnp.mean(h, axis=1, keepdims=True)
    var = jnp.mean((h - mu) ** 2, axis=1, keepdims=True)
    flat = (h - mu) / jnp.sqrt(var + 1e-5) * g[...] + beta[...]

    x2 = jnp.sum(flat * flat, axis=1, keepdims=True)
    m2 = jnp.dot(flat * (-2.0), cbt[...], preferred_element_type=jnp.float32)
    d = (x2 + e2_scr[...]) + m2
    idxs = jnp.argmin(d, axis=1)[:, None]
    idx_out[0, :, :] = idxs
    iota = lax.broadcasted_iota(jnp.int32, (_R1, _K), 1)
    enc_out[...] = (iota == idxs).astype(jnp.float32)


def _front_body_b(feat, w1, b1, w2, b2, g, beta, cbt, enc_prev, idx_out,
                  enc_out, e2_scr):
    _front_body(feat, w1, b1, w2, b2, g, beta, cbt, idx_out, enc_out, e2_scr)


def _front_call(part):
    off = part * _GH
    body = _front_body if part == 0 else _front_body_b
    in_specs = [
        pl.BlockSpec((_R1, _D), lambda i: (i + off, 0)),
        pl.BlockSpec((_D, _D), lambda i: (0, 0)),
        pl.BlockSpec((1, _D), lambda i: (0, 0)),
        pl.BlockSpec((_D, _CD), lambda i: (0, 0)),
        pl.BlockSpec((1, _CD), lambda i: (0, 0)),
        pl.BlockSpec((1, _CD), lambda i: (0, 0)),
        pl.BlockSpec((1, _CD), lambda i: (0, 0)),
        pl.BlockSpec((_CD, _K), lambda i: (0, 0)),
    ]
    kwargs = {}
    if part > 0:
        in_specs.append(pl.BlockSpec(memory_space=pl.ANY))
        kwargs["input_output_aliases"] = {8: 1}
    return pl.pallas_call(
        body,
        grid=(_GH,),
        in_specs=in_specs,
        out_specs=[
            pl.BlockSpec((1, _R1, 1), lambda i: (i, 0, 0)),
            pl.BlockSpec((_R1, _K), lambda i: (i + off, 0)),
        ],
        out_shape=[
            jax.ShapeDtypeStruct((_GH, _R1, 1), jnp.int32),
            jax.ShapeDtypeStruct((_N, _K), jnp.float32),
        ],
        scratch_shapes=[pltpu.VMEM((1, _K), jnp.float32)],
        compiler_params=pltpu.CompilerParams(
            dimension_semantics=("arbitrary",)),
        **kwargs,
    )


_fronts = [_front_call(p) for p in range(_S)]

_NC, _NS = 2, 16
_NW = _NC * _NS
_RPW = _H // _NW
_CH = 128


def _gather_body(cb_hbm, idx_hbm, out_hbm, idx_v, rows_v, gsem, ssem):
    wid = lax.axis_index("s") * _NC + lax.axis_index("c")
    base = wid * _RPW
    pltpu.sync_copy(idx_hbm.at[pl.ds(base, _RPW)], idx_v)

    def g(ch, buf):
        return pltpu.async_copy(
            cb_hbm.at[idx_v.at[pl.ds(ch * _CH, _CH)]], rows_v.at[buf], gsem)

    def s(ch, buf):
        return pltpu.async_copy(
            rows_v.at[buf], out_hbm.at[pl.ds(base + ch * _CH, _CH)], ssem)

    g0 = g(0, 0)
    g0.wait()
    s0 = s(0, 0)
    s0.wait()


@functools.cache
def _build_gather():
    return functools.partial(
        pl.kernel,
        out_type=jax.ShapeDtypeStruct((_H, _CD), jnp.float32),
        mesh=plsc.VectorSubcoreMesh(core_axis_name="c", subcore_axis_name="s"),
        scratch_types=[
            pltpu.VMEM((_RPW,), jnp.int32),
            pltpu.VMEM((1, _CH, _CD), jnp.float32),
            pltpu.SemaphoreType.DMA,
            pltpu.SemaphoreType.DMA,
        ],
    )(_gather_body)


def _gather(cb, idx):
    return _build_gather()(cb, idx)

_R3 = 1024
_G3 = _H // _R3


def _back_body(qr, wo1, bo1, wo2, bo2, g, beta, out):
    h = jnp.maximum(jnp.dot(qr[...], wo1[...], preferred_element_type=jnp.float32)
                    + bo1[...], 0.0)
    h = jnp.dot(h, wo2[...], preferred_element_type=jnp.float32) + bo2[...]
    mu = jnp.mean(h, axis=1, keepdims=True)
    var = jnp.mean((h - mu) ** 2, axis=1, keepdims=True)
    out[...] = (h - mu) / jnp.sqrt(var + 1e-5) * g[...] + beta[...]


def _back_body_b(qr, wo1, bo1, wo2, bo2, g, beta, q_prev, out):
    _back_body(qr, wo1, bo1, wo2, bo2, g, beta, out)


def _back_call(part):
    off = part * _G3
    body = _back_body if part == 0 else _back_body_b
    in_specs = [
        pl.BlockSpec((_R3, _CD), lambda i: (i, 0)),
        pl.BlockSpec((_CD, _D), lambda i: (0, 0)),
        pl.BlockSpec((1, _D), lambda i: (0, 0)),
        pl.BlockSpec((_D, _D), lambda i: (0, 0)),
        pl.BlockSpec((1, _D), lambda i: (0, 0)),
        pl.BlockSpec((1, _D), lambda i: (0, 0)),
        pl.BlockSpec((1, _D), lambda i: (0, 0)),
    ]
    kwargs = {}
    if part > 0:
        in_specs.append(pl.BlockSpec(memory_space=pl.ANY))
        kwargs["input_output_aliases"] = {7: 0}
    return pl.pallas_call(
        body,
        grid=(_G3,),
        in_specs=in_specs,
        out_specs=pl.BlockSpec((_R3, _D), lambda i: (i + off, 0)),
        out_shape=jax.ShapeDtypeStruct((_N, _D), jnp.float32),
        compiler_params=pltpu.CompilerParams(
            dimension_semantics=("arbitrary",)),
        **kwargs,
    )


_backs = [_back_call(p) for p in range(_S)]


def kernel(features, W_in1, b_in1, W_in2, b_in2, g_nin, beta_nin, codebook,
           W_out1, b_out1, W_out2, b_out2, g_nout, beta_nout):
    feat = features.reshape(_N, _D)
    cbt = codebook.T
    wargs = (W_in1, b_in1.reshape(1, -1), W_in2, b_in2.reshape(1, -1),
             g_nin.reshape(1, -1), beta_nin.reshape(1, -1), cbt)
    oargs = (W_out1, b_out1.reshape(1, -1), W_out2, b_out2.reshape(1, -1),
             g_nout.reshape(1, -1), beta_nout.reshape(1, -1))
    idxs, enc = [], None
    for p in range(_S):
        extra = () if p == 0 else (enc,)
        idx3, enc = _fronts[p](feat, *wargs, *extra)
        idxs.append(idx3.reshape(_H))
    qrs = [_gather(codebook, ix) for ix in idxs]
    q = None
    for p in range(_S):
        extra = () if p == 0 else (q,)
        q = _backs[p](qrs[p], *oargs, *extra)
    idx_flat = jnp.concatenate(idxs)
    return q.reshape(_B, _T, _D), idx_flat.reshape(-1, 1), enc


# speedup vs baseline: 1.0707x; 1.0707x over previous
import functools

import jax
import jax.numpy as jnp
from jax import lax
from jax.experimental import pallas as pl
from jax.experimental.pallas import tpu as pltpu
from jax.experimental.pallas import tpu_sc as plsc

_B, _T, _D, _CD, _K = 16, 1024, 768, 256, 8192
_N = _B * _T
_PARTS = (12288, 4096)
_R1 = 512


def _front_body(feat, w1, b1, w2, b2, g, beta, cbt, idx_out, enc_out, e2_scr):
    @pl.when(pl.program_id(0) == 0)
    def _():
        c = cbt[...]
        e2_scr[...] = jnp.sum(c * c, axis=0, keepdims=True)

    x = feat[...]
    h = jnp.maximum(jnp.dot(x, w1[...], preferred_element_type=jnp.float32)
                    + b1[...], 0.0)
    h = jnp.dot(h, w2[...], preferred_element_type=jnp.float32) + b2[...]
    mu = jnp.mean(h, axis=1, keepdims=True)
    var = jnp.mean((h - mu) ** 2, axis=1, keepdims=True)
    flat = (h - mu) / jnp.sqrt(var + 1e-5) * g[...] + beta[...]

    x2 = jnp.sum(flat * flat, axis=1, keepdims=True)
    m2 = jnp.dot(flat * (-2.0), cbt[...], preferred_element_type=jnp.float32)
    d = (x2 + e2_scr[...]) + m2
    idxs = jnp.argmin(d, axis=1)[:, None]
    idx_out[0, :, :] = idxs
    iota = lax.broadcasted_iota(jnp.int32, (_R1, _K), 1)
    enc_out[...] = (iota == idxs).astype(jnp.float32)


def _front_body_b(feat, w1, b1, w2, b2, g, beta, cbt, enc_prev, idx_out,
                  enc_out, e2_scr):
    _front_body(feat, w1, b1, w2, b2, g, beta, cbt, idx_out, enc_out, e2_scr)


def _front_call(part):
    off = sum(_PARTS[:part]) // _R1
    grid = _PARTS[part] // _R1
    body = _front_body if part == 0 else _front_body_b
    in_specs = [
        pl.BlockSpec((_R1, _D), lambda i: (i + off, 0)),
        pl.BlockSpec((_D, _D), lambda i: (0, 0)),
        pl.BlockSpec((1, _D), lambda i: (0, 0)),
        pl.BlockSpec((_D, _CD), lambda i: (0, 0)),
        pl.BlockSpec((1, _CD), lambda i: (0, 0)),
        pl.BlockSpec((1, _CD), lambda i: (0, 0)),
        pl.BlockSpec((1, _CD), lambda i: (0, 0)),
        pl.BlockSpec((_CD, _K), lambda i: (0, 0)),
    ]
    kwargs = {}
    if part > 0:
        in_specs.append(pl.BlockSpec(memory_space=pl.ANY))
        kwargs["input_output_aliases"] = {8: 1}
    return pl.pallas_call(
        body,
        grid=(grid,),
        in_specs=in_specs,
        out_specs=[
            pl.BlockSpec((1, _R1, 1), lambda i: (i, 0, 0)),
            pl.BlockSpec((_R1, _K), lambda i: (i + off, 0)),
        ],
        out_shape=[
            jax.ShapeDtypeStruct((grid, _R1, 1), jnp.int32),
            jax.ShapeDtypeStruct((_N, _K), jnp.float32),
        ],
        scratch_shapes=[pltpu.VMEM((1, _K), jnp.float32)],
        compiler_params=pltpu.CompilerParams(
            dimension_semantics=("arbitrary",)),
        **kwargs,
    )


_fronts = [_front_call(p) for p in range(len(_PARTS))]

_NC, _NS = 2, 16
_NW = _NC * _NS
_CH = 128


def _make_gather_body(rpw):
    nch = rpw // _CH

    def body(cb_hbm, idx_hbm, out_hbm, idx_v, rows_v, gsem, ssem):
        wid = lax.axis_index("s") * _NC + lax.axis_index("c")
        base = wid * rpw
        pltpu.sync_copy(idx_hbm.at[pl.ds(base, rpw)], idx_v)

        def g(ch, buf):
            return pltpu.async_copy(
                cb_hbm.at[idx_v.at[pl.ds(ch * _CH, _CH)]], rows_v.at[buf],
                gsem)

        def s(ch, buf):
            return pltpu.async_copy(
                rows_v.at[buf], out_hbm.at[pl.ds(base + ch * _CH, _CH)], ssem)

        cps = [g(ch, ch % 2) for ch in range(min(2, nch))]
        for ch in range(nch):
            cps[ch].wait()
            st = s(ch, ch % 2)
            st.wait()
            if ch + 2 < nch:
                cps.append(g(ch + 2, ch % 2))

    return body


@functools.cache
def _build_gather(n_rows):
    rpw = n_rows // _NW
    return functools.partial(
        pl.kernel,
        out_type=jax.ShapeDtypeStruct((n_rows, _CD), jnp.float32),
        mesh=plsc.VectorSubcoreMesh(core_axis_name="c", subcore_axis_name="s"),
        scratch_types=[
            pltpu.VMEM((rpw,), jnp.int32),
            pltpu.VMEM((2, _CH, _CD), jnp.float32),
            pltpu.SemaphoreType.DMA,
            pltpu.SemaphoreType.DMA,
        ],
    )(_make_gather_body(rpw))


def _gather(cb, idx):
    return _build_gather(idx.shape[0])(cb, idx)

_R3 = 1024


def _back_body(qr, wo1, bo1, wo2, bo2, g, beta, out):
    h = jnp.maximum(jnp.dot(qr[...], wo1[...], preferred_element_type=jnp.float32)
                    + bo1[...], 0.0)
    h = jnp.dot(h, wo2[...], preferred_element_type=jnp.float32) + bo2[...]
    mu = jnp.mean(h, axis=1, keepdims=True)
    var = jnp.mean((h - mu) ** 2, axis=1, keepdims=True)
    out[...] = (h - mu) / jnp.sqrt(var + 1e-5) * g[...] + beta[...]


def _back_body_b(qr, wo1, bo1, wo2, bo2, g, beta, q_prev, out):
    _back_body(qr, wo1, bo1, wo2, bo2, g, beta, out)


def _back_call(part):
    off = sum(_PARTS[:part]) // _R3
    grid = _PARTS[part] // _R3
    body = _back_body if part == 0 else _back_body_b
    in_specs = [
        pl.BlockSpec((_R3, _CD), lambda i: (i, 0)),
        pl.BlockSpec((_CD, _D), lambda i: (0, 0)),
        pl.BlockSpec((1, _D), lambda i: (0, 0)),
        pl.BlockSpec((_D, _D), lambda i: (0, 0)),
        pl.BlockSpec((1, _D), lambda i: (0, 0)),
        pl.BlockSpec((1, _D), lambda i: (0, 0)),
        pl.BlockSpec((1, _D), lambda i: (0, 0)),
    ]
    kwargs = {}
    if part > 0:
        in_specs.append(pl.BlockSpec(memory_space=pl.ANY))
        kwargs["input_output_aliases"] = {7: 0}
    return pl.pallas_call(
        body,
        grid=(grid,),
        in_specs=in_specs,
        out_specs=pl.BlockSpec((_R3, _D), lambda i: (i + off, 0)),
        out_shape=jax.ShapeDtypeStruct((_N, _D), jnp.float32),
        compiler_params=pltpu.CompilerParams(
            dimension_semantics=("arbitrary",)),
        **kwargs,
    )


_backs = [_back_call(p) for p in range(len(_PARTS))]


def kernel(features, W_in1, b_in1, W_in2, b_in2, g_nin, beta_nin, codebook,
           W_out1, b_out1, W_out2, b_out2, g_nout, beta_nout):
    feat = features.reshape(_N, _D)
    cbt = codebook.T
    wargs = (W_in1, b_in1.reshape(1, -1), W_in2, b_in2.reshape(1, -1),
             g_nin.reshape(1, -1), beta_nin.reshape(1, -1), cbt)
    oargs = (W_out1, b_out1.reshape(1, -1), W_out2, b_out2.reshape(1, -1),
             g_nout.reshape(1, -1), beta_nout.reshape(1, -1))
    idxs, enc = [], None
    for p in range(len(_PARTS)):
        extra = () if p == 0 else (enc,)
        idx3, enc = _fronts[p](feat, *wargs, *extra)
        idxs.append(idx3.reshape(_PARTS[p]))
    qrs = [_gather(codebook, ix) for ix in idxs]
    q = None
    for p in range(len(_PARTS)):
        extra = () if p == 0 else (q,)
        q = _backs[p](qrs[p], *oargs, *extra)
    idx_flat = jnp.concatenate(idxs)
    return q.reshape(_B, _T, _D), idx_flat.reshape(-1, 1), enc


# speedup vs baseline: 1.0830x; 1.0114x over previous
import functools

import jax
import jax.numpy as jnp
from jax import lax
from jax.experimental import pallas as pl
from jax.experimental.pallas import tpu as pltpu
from jax.experimental.pallas import tpu_sc as plsc

_B, _T, _D, _CD, _K = 16, 1024, 768, 256, 8192
_N = _B * _T
_PARTS = (8192, 8192)
_R1 = 512


def _front_body(feat, w1, b1, w2, b2, g, beta, cbt, idx_out, enc_out, e2_scr):
    @pl.when(pl.program_id(0) == 0)
    def _():
        c = cbt[...]
        e2_scr[...] = jnp.sum(c * c, axis=0, keepdims=True)

    x = feat[...]
    h = jnp.maximum(jnp.dot(x, w1[...], preferred_element_type=jnp.float32)
                    + b1[...], 0.0)
    h = jnp.dot(h, w2[...], preferred_element_type=jnp.float32) + b2[...]
    mu = jnp.mean(h, axis=1, keepdims=True)
    var = jnp.mean((h - mu) ** 2, axis=1, keepdims=True)
    flat = (h - mu) / jnp.sqrt(var + 1e-5) * g[...] + beta[...]

    x2 = jnp.sum(flat * flat, axis=1, keepdims=True)
    m2 = jnp.dot(flat * (-2.0), cbt[...], preferred_element_type=jnp.float32)
    d = (x2 + e2_scr[...]) + m2
    idxs = jnp.argmin(d, axis=1)[:, None]
    idx_out[0, :, :] = idxs
    iota = lax.broadcasted_iota(jnp.int32, (_R1, _K), 1)
    enc_out[...] = (iota == idxs).astype(jnp.float32)


def _front_body_b(feat, w1, b1, w2, b2, g, beta, cbt, enc_prev, idx_out,
                  enc_out, e2_scr):
    _front_body(feat, w1, b1, w2, b2, g, beta, cbt, idx_out, enc_out, e2_scr)


def _front_call(part):
    off = sum(_PARTS[:part]) // _R1
    grid = _PARTS[part] // _R1
    body = _front_body if part == 0 else _front_body_b
    in_specs = [
        pl.BlockSpec((_R1, _D), lambda i: (i + off, 0)),
        pl.BlockSpec((_D, _D), lambda i: (0, 0)),
        pl.BlockSpec((1, _D), lambda i: (0, 0)),
        pl.BlockSpec((_D, _CD), lambda i: (0, 0)),
        pl.BlockSpec((1, _CD), lambda i: (0, 0)),
        pl.BlockSpec((1, _CD), lambda i: (0, 0)),
        pl.BlockSpec((1, _CD), lambda i: (0, 0)),
        pl.BlockSpec((_CD, _K), lambda i: (0, 0)),
    ]
    kwargs = {}
    if part > 0:
        in_specs.append(pl.BlockSpec(memory_space=pl.ANY))
        kwargs["input_output_aliases"] = {8: 1}
    return pl.pallas_call(
        body,
        grid=(grid,),
        in_specs=in_specs,
        out_specs=[
            pl.BlockSpec((1, _R1, 1), lambda i: (i, 0, 0)),
            pl.BlockSpec((_R1, _K), lambda i: (i + off, 0)),
        ],
        out_shape=[
            jax.ShapeDtypeStruct((grid, _R1, 1), jnp.int32),
            jax.ShapeDtypeStruct((_N, _K), jnp.float32),
        ],
        scratch_shapes=[pltpu.VMEM((1, _K), jnp.float32)],
        compiler_params=pltpu.CompilerParams(
            dimension_semantics=("arbitrary",)),
        **kwargs,
    )


_fronts = [_front_call(p) for p in range(len(_PARTS))]

_NC, _NS = 2, 16
_NW = _NC * _NS
_CH = 128


def _make_gather_body(rpw):
    nch = rpw // _CH

    def body(cb_hbm, idx_hbm, out_hbm, idx_v, rows_v, gsem, ssem):
        wid = lax.axis_index("s") * _NC + lax.axis_index("c")
        base = wid * rpw
        pltpu.sync_copy(idx_hbm.at[pl.ds(base, rpw)], idx_v)

        def g(ch, buf):
            return pltpu.async_copy(
                cb_hbm.at[idx_v.at[pl.ds(ch * _CH, _CH)]], rows_v.at[buf],
                gsem)

        def s(ch, buf):
            return pltpu.async_copy(
                rows_v.at[buf], out_hbm.at[pl.ds(base + ch * _CH, _CH)], ssem)

        cps = [g(ch, ch % 2) for ch in range(min(2, nch))]
        for ch in range(nch):
            cps[ch].wait()
            st = s(ch, ch % 2)
            st.wait()
            if ch + 2 < nch:
                cps.append(g(ch + 2, ch % 2))

    return body


@functools.cache
def _build_gather(n_rows):
    rpw = n_rows // _NW
    return functools.partial(
        pl.kernel,
        out_type=jax.ShapeDtypeStruct((n_rows, _CD), jnp.float32),
        mesh=plsc.VectorSubcoreMesh(core_axis_name="c", subcore_axis_name="s"),
        scratch_types=[
            pltpu.VMEM((rpw,), jnp.int32),
            pltpu.VMEM((2, _CH, _CD), jnp.float32),
            pltpu.SemaphoreType.DMA,
            pltpu.SemaphoreType.DMA,
        ],
    )(_make_gather_body(rpw))


def _gather(cb, idx):
    return _build_gather(idx.shape[0])(cb, idx)

_R3 = 1024


def _back_body(qr, wo1, bo1, wo2, bo2, g, beta, out):
    h = jnp.maximum(jnp.dot(qr[...], wo1[...], preferred_element_type=jnp.float32)
                    + bo1[...], 0.0)
    h = jnp.dot(h, wo2[...], preferred_element_type=jnp.float32) + bo2[...]
    mu = jnp.mean(h, axis=1, keepdims=True)
    var = jnp.mean((h - mu) ** 2, axis=1, keepdims=True)
    out[...] = (h - mu) / jnp.sqrt(var + 1e-5) * g[...] + beta[...]


def _back_body_b(qr, wo1, bo1, wo2, bo2, g, beta, q_prev, out):
    _back_body(qr, wo1, bo1, wo2, bo2, g, beta, out)


def _back_call(part):
    off = sum(_PARTS[:part]) // _R3
    grid = _PARTS[part] // _R3
    body = _back_body if part == 0 else _back_body_b
    in_specs = [
        pl.BlockSpec((_R3, _CD), lambda i: (i, 0)),
        pl.BlockSpec((_CD, _D), lambda i: (0, 0)),
        pl.BlockSpec((1, _D), lambda i: (0, 0)),
        pl.BlockSpec((_D, _D), lambda i: (0, 0)),
        pl.BlockSpec((1, _D), lambda i: (0, 0)),
        pl.BlockSpec((1, _D), lambda i: (0, 0)),
        pl.BlockSpec((1, _D), lambda i: (0, 0)),
    ]
    kwargs = {}
    if part > 0:
        in_specs.append(pl.BlockSpec(memory_space=pl.ANY))
        kwargs["input_output_aliases"] = {7: 0}
    return pl.pallas_call(
        body,
        grid=(grid,),
        in_specs=in_specs,
        out_specs=pl.BlockSpec((_R3, _D), lambda i: (i + off, 0)),
        out_shape=jax.ShapeDtypeStruct((_N, _D), jnp.float32),
        compiler_params=pltpu.CompilerParams(
            dimension_semantics=("arbitrary",)),
        **kwargs,
    )


_backs = [_back_call(p) for p in range(len(_PARTS))]


def kernel(features, W_in1, b_in1, W_in2, b_in2, g_nin, beta_nin, codebook,
           W_out1, b_out1, W_out2, b_out2, g_nout, beta_nout):
    feat = features.reshape(_N, _D)
    cbt = codebook.T
    wargs = (W_in1, b_in1.reshape(1, -1), W_in2, b_in2.reshape(1, -1),
             g_nin.reshape(1, -1), beta_nin.reshape(1, -1), cbt)
    oargs = (W_out1, b_out1.reshape(1, -1), W_out2, b_out2.reshape(1, -1),
             g_nout.reshape(1, -1), beta_nout.reshape(1, -1))
    idxs, enc = [], None
    for p in range(len(_PARTS)):
        extra = () if p == 0 else (enc,)
        idx3, enc = _fronts[p](feat, *wargs, *extra)
        idxs.append(idx3.reshape(_PARTS[p]))
    qrs = [_gather(codebook, ix) for ix in idxs]
    q = None
    for p in range(len(_PARTS)):
        extra = () if p == 0 else (q,)
        q = _backs[p](qrs[p], *oargs, *extra)
    idx_flat = jnp.concatenate(idxs)
    return q.reshape(_B, _T, _D), idx_flat.reshape(-1, 1), enc
